# baseline (speedup 1.0000x reference)
import functools

import jax
import jax.numpy as jnp
from jax import lax
from jax.experimental import pallas as pl
from jax.experimental.pallas import tpu as pltpu
from jax.experimental.pallas import tpu_sc as plsc

VOCAB = 1000000
EMBED_DIM = 64
MAXLEN = 200
BATCH = 4096
OUTPUT_SIZE = 10

NUM_CORES = 2
NUM_SUBCORES = 16
NUM_WORKERS = NUM_CORES * NUM_SUBCORES
ROWS_PER_WORKER = BATCH // NUM_WORKERS
CHUNK = MAXLEN // 2
CHUNKS_PER_WORKER = 2 * ROWS_PER_WORKER
LANES = 16
VECS_PER_ROW = EMBED_DIM // LANES
UNROLL = 4


def _pool_body(idx_hbm, table_hbm, out_hbm, idx_v, rows_v, pooled_v, sem0, sem1):
    wid = lax.axis_index("s") * NUM_CORES + lax.axis_index("c")
    cbase = wid * CHUNKS_PER_WORKER

    pltpu.sync_copy(idx_hbm.at[pl.ds(cbase, CHUNKS_PER_WORKER), :], idx_v)

    sems = (sem0, sem1)

    def issue(r, slot):
        c0 = 2 * r
        pltpu.async_copy(table_hbm.at[idx_v.at[c0]],
                         rows_v.at[slot, pl.ds(0, CHUNK)], sems[slot])
        pltpu.async_copy(table_hbm.at[idx_v.at[c0 + 1]],
                         rows_v.at[slot, pl.ds(CHUNK, CHUNK)], sems[slot])

    def wait_slot(slot):
        pltpu.make_async_copy(table_hbm.at[pl.ds(0, MAXLEN), :],
                              rows_v.at[slot], sems[slot]).wait()

    def accum(r, slot):
        def step(i, acc):
            base = i * UNROLL
            acc = list(acc)
            for u in range(UNROLL):
                row = base + u
                for d in range(VECS_PER_ROW):
                    acc[d] = acc[d] + rows_v[slot, row, pl.ds(d * LANES, LANES)]
            return tuple(acc)

        zero = jnp.zeros((LANES,), jnp.float32)
        acc = lax.fori_loop(0, MAXLEN // UNROLL, step, (zero,) * VECS_PER_ROW)
        scale = jnp.float32(1.0 / MAXLEN)
        for d in range(VECS_PER_ROW):
            pooled_v[r, pl.ds(d * LANES, LANES)] = acc[d] * scale

    issue(0, 0)
    issue(1, 1)

    def outer(i, carry):
        r0 = 2 * i
        for slot in range(2):
            r = r0 + slot
            wait_slot(slot)
            accum(r, slot)

            @pl.when(r + 2 < ROWS_PER_WORKER)
            def _():
                issue(r + 2, slot)
        return carry

    lax.fori_loop(0, ROWS_PER_WORKER // 2, outer, 0)

    pltpu.sync_copy(pooled_v,
                    out_hbm.at[pl.ds(wid * ROWS_PER_WORKER, ROWS_PER_WORKER), :])


_pool_call = pl.kernel(
    _pool_body,
    out_type=jax.ShapeDtypeStruct((BATCH, EMBED_DIM), jnp.float32),
    mesh=plsc.VectorSubcoreMesh(core_axis_name="c", subcore_axis_name="s",
                                num_cores=NUM_CORES, num_subcores=NUM_SUBCORES),
    scratch_types=[
        pltpu.VMEM((CHUNKS_PER_WORKER, CHUNK), jnp.int32),
        pltpu.VMEM((2, MAXLEN, EMBED_DIM), jnp.float32),
        pltpu.VMEM((ROWS_PER_WORKER, EMBED_DIM), jnp.float32),
        pltpu.SemaphoreType.DMA,
        pltpu.SemaphoreType.DMA,
    ],
    compiler_params=pltpu.CompilerParams(use_tc_tiling_on_sc=False),
)


def _dense_softmax_body(pooled_ref, w_ref, b_ref, out_ref):
    logits = jnp.dot(pooled_ref[...], w_ref[...],
                     preferred_element_type=jnp.float32) + b_ref[...]
    m = jnp.max(logits, axis=-1, keepdims=True)
    e = jnp.exp(logits - m)
    out_ref[...] = e / jnp.sum(e, axis=-1, keepdims=True)


_dense_call = pl.pallas_call(
    _dense_softmax_body,
    out_shape=jax.ShapeDtypeStruct((BATCH, 128), jnp.float32),
)


def kernel(indices, table, W, b):
    idx2 = indices.astype(jnp.int32).reshape(2 * BATCH, CHUNK)
    pooled = _pool_call(idx2, table)
    w_pad = jnp.zeros((EMBED_DIM, 128), jnp.float32).at[:, :OUTPUT_SIZE].set(W)
    b_pad = jnp.full((1, 128), -1e30, jnp.float32).at[0, :OUTPUT_SIZE].set(b)
    probs_pad = _dense_call(pooled, w_pad, b_pad)
    return probs_pad[:, :OUTPUT_SIZE]


# speedup vs baseline: 1.0050x; 1.0050x over previous
import functools

import jax
import jax.numpy as jnp
from jax import lax
from jax.experimental import pallas as pl
from jax.experimental.pallas import tpu as pltpu
from jax.experimental.pallas import tpu_sc as plsc

VOCAB = 1000000
EMBED_DIM = 64
MAXLEN = 200
BATCH = 4096
OUTPUT_SIZE = 10

NUM_CORES = 2
NUM_SUBCORES = 16
NUM_WORKERS = NUM_CORES * NUM_SUBCORES
ROWS_PER_WORKER = BATCH // NUM_WORKERS
CHUNK_A = 128
CHUNK_B = MAXLEN - CHUNK_A
LANES = 16
VECS_PER_ROW = EMBED_DIM // LANES
UNROLL = 4


def _pool_body(idx_hbm, table_hbm, out_hbm, idx_v, rows_v, pooled_v, sem0, sem1):
    wid = lax.axis_index("s") * NUM_CORES + lax.axis_index("c")
    rbase = wid * ROWS_PER_WORKER

    pltpu.sync_copy(idx_hbm.at[pl.ds(rbase, ROWS_PER_WORKER), :], idx_v)

    sems = (sem0, sem1)

    def issue(r, slot):
        pltpu.async_copy(table_hbm.at[idx_v.at[r, pl.ds(0, CHUNK_A)]],
                         rows_v.at[slot, pl.ds(0, CHUNK_A)], sems[slot])
        pltpu.async_copy(table_hbm.at[idx_v.at[r, pl.ds(CHUNK_A, CHUNK_B)]],
                         rows_v.at[slot, pl.ds(CHUNK_A, CHUNK_B)], sems[slot])

    def wait_slot(slot):
        pltpu.make_async_copy(table_hbm.at[pl.ds(0, MAXLEN), :],
                              rows_v.at[slot], sems[slot]).wait()

    def accum(r, slot):
        def step(i, acc):
            base = i * UNROLL
            acc = list(acc)
            for u in range(UNROLL):
                row = base + u
                for d in range(VECS_PER_ROW):
                    acc[d] = acc[d] + rows_v[slot, row, pl.ds(d * LANES, LANES)]
            return tuple(acc)

        zero = jnp.zeros((LANES,), jnp.float32)
        acc = lax.fori_loop(0, MAXLEN // UNROLL, step, (zero,) * VECS_PER_ROW)
        scale = jnp.float32(1.0 / MAXLEN)
        for d in range(VECS_PER_ROW):
            pooled_v[r, pl.ds(d * LANES, LANES)] = acc[d] * scale

    issue(0, 0)
    issue(1, 1)

    def outer(i, carry):
        r0 = 2 * i
        for slot in range(2):
            r = r0 + slot
            wait_slot(slot)
            accum(r, slot)

            @pl.when(r + 2 < ROWS_PER_WORKER)
            def _():
                issue(r + 2, slot)
        return carry

    lax.fori_loop(0, ROWS_PER_WORKER // 2, outer, 0)

    pltpu.sync_copy(pooled_v,
                    out_hbm.at[pl.ds(wid * ROWS_PER_WORKER, ROWS_PER_WORKER), :])


_pool_call = pl.kernel(
    _pool_body,
    out_type=jax.ShapeDtypeStruct((BATCH, EMBED_DIM), jnp.float32),
    mesh=plsc.VectorSubcoreMesh(core_axis_name="c", subcore_axis_name="s",
                                num_cores=NUM_CORES, num_subcores=NUM_SUBCORES),
    scratch_types=[
        pltpu.VMEM((ROWS_PER_WORKER, MAXLEN), jnp.int32),
        pltpu.VMEM((2, MAXLEN, EMBED_DIM), jnp.float32),
        pltpu.VMEM((ROWS_PER_WORKER, EMBED_DIM), jnp.float32),
        pltpu.SemaphoreType.DMA,
        pltpu.SemaphoreType.DMA,
    ],
    compiler_params=pltpu.CompilerParams(use_tc_tiling_on_sc=False),
)


def _dense_softmax_body(pooled_ref, w_ref, b_ref, out_ref):
    logits = jnp.dot(pooled_ref[...], w_ref[...],
                     preferred_element_type=jnp.float32) + b_ref[...]
    m = jnp.max(logits, axis=-1, keepdims=True)
    e = jnp.exp(logits - m)
    out_ref[...] = e / jnp.sum(e, axis=-1, keepdims=True)


_dense_call = pl.pallas_call(
    _dense_softmax_body,
    out_shape=jax.ShapeDtypeStruct((BATCH, 128), jnp.float32),
)


def kernel(indices, table, W, b):
    pooled = _pool_call(indices.astype(jnp.int32), table)
    w_pad = jnp.zeros((EMBED_DIM, 128), jnp.float32).at[:, :OUTPUT_SIZE].set(W)
    b_pad = jnp.full((1, 128), -1e30, jnp.float32).at[0, :OUTPUT_SIZE].set(b)
    probs_pad = _dense_call(pooled, w_pad, b_pad)
    return probs_pad[:, :OUTPUT_SIZE]
